# Initial kernel scaffold; baseline (speedup 1.0000x reference)
import jax
import jax.numpy as jnp
from jax.experimental import pallas as pl


def kernel(inputs, labels, W, b):
    raise NotImplementedError("write your pallas kernel here")


# speedup vs baseline: 1.2551x; 1.2551x over previous
import functools

import jax
import jax.numpy as jnp
from jax import lax
from jax.experimental import pallas as pl
from jax.experimental.pallas import tpu as pltpu
from jax.experimental.pallas import tpu_sc as plsc

NUM_CLASSES = 200000
NUM_SAMPLED = 1000
BATCH = 4096
DIM = 128
S_PAD = 1024

_NC = 2
_NS = 16
_NW = _NC * _NS

_LAB_PER_W = BATCH // _NW
_SMP_PER_W = S_PAD // _NW

_BB = 512


_B_ROWS = (NUM_CLASSES + DIM - 1) // DIM


@functools.cache
def _sc_gather_kernel():
    mesh = plsc.VectorSubcoreMesh(
        core_axis_name="c", subcore_axis_name="s",
        num_cores=_NC, num_subcores=_NS)

    @functools.partial(
        pl.kernel,
        out_type=[
            jax.ShapeDtypeStruct((BATCH, DIM), jnp.float32),
            jax.ShapeDtypeStruct((S_PAD, DIM), jnp.float32),
            jax.ShapeDtypeStruct((BATCH, DIM), jnp.float32),
            jax.ShapeDtypeStruct((S_PAD, DIM), jnp.float32),
        ],
        mesh=mesh,
        scratch_types=[
            pltpu.VMEM((_LAB_PER_W,), jnp.int32),
            pltpu.VMEM((_SMP_PER_W,), jnp.int32),
            pltpu.VMEM((_LAB_PER_W,), jnp.int32),
            pltpu.VMEM((_SMP_PER_W,), jnp.int32),
            pltpu.VMEM((_LAB_PER_W, DIM), jnp.float32),
            pltpu.VMEM((_SMP_PER_W, DIM), jnp.float32),
            pltpu.VMEM((_LAB_PER_W, DIM), jnp.float32),
            pltpu.VMEM((_SMP_PER_W, DIM), jnp.float32),
            pltpu.SemaphoreType.DMA,
            pltpu.SemaphoreType.DMA,
            pltpu.SemaphoreType.DMA,
            pltpu.SemaphoreType.DMA,
        ],
    )
    def sc_gather(w_hbm, brs_hbm, labels_hbm, sampled_hbm,
                  tw_out, sw_out, tbr_out, sbr_out,
                  lab_v, smp_v, lrid_v, srid_v, tw_v, sw_v, lbrow_v, sbrow_v,
                  s1, s2, s3, s4):
        wid = lax.axis_index("s") * _NC + lax.axis_index("c")
        lbase = wid * _LAB_PER_W
        sbase = wid * _SMP_PER_W
        pltpu.sync_copy(labels_hbm.at[pl.ds(lbase, _LAB_PER_W)], lab_v)
        pltpu.sync_copy(sampled_hbm.at[pl.ds(sbase, _SMP_PER_W)], smp_v)
        for k in range(_LAB_PER_W // 16):
            chunk = lab_v[pl.ds(16 * k, 16)]
            lrid_v[pl.ds(16 * k, 16)] = lax.shift_right_logical(chunk, 7)
        for k in range(_SMP_PER_W // 16):
            chunk = smp_v[pl.ds(16 * k, 16)]
            srid_v[pl.ds(16 * k, 16)] = lax.shift_right_logical(chunk, 7)
        c1 = pltpu.async_copy(w_hbm.at[lab_v], tw_v, s1)
        c2 = pltpu.async_copy(w_hbm.at[smp_v], sw_v, s2)
        c3 = pltpu.async_copy(brs_hbm.at[lrid_v], lbrow_v, s3)
        c4 = pltpu.async_copy(brs_hbm.at[srid_v], sbrow_v, s4)
        c2.wait()
        pltpu.sync_copy(sw_v, sw_out.at[pl.ds(sbase, _SMP_PER_W)])
        c4.wait()
        pltpu.sync_copy(sbrow_v, sbr_out.at[pl.ds(sbase, _SMP_PER_W)])
        c1.wait()
        pltpu.sync_copy(tw_v, tw_out.at[pl.ds(lbase, _LAB_PER_W)])
        c3.wait()
        pltpu.sync_copy(lbrow_v, tbr_out.at[pl.ds(lbase, _LAB_PER_W)])

    return sc_gather


def _log1m(p):
    return -p * (1.0 + p * (0.5 + p * (1.0 / 3.0 + p * (
        0.25 + p * (0.2 + p * (1.0 / 6.0))))))


def _expm1_neg(y):
    small = y * (1.0 + y * (0.5 + y * (1.0 / 6.0)))
    return jnp.where(y > -0.02, small, jnp.exp(y) - 1.0)


def _tc_body(x_ref, tw_ref, tbr_ref, lab_ref, sw_ref, sbr_ref, slane_ref,
             cadj_ref, sid_ref, out_ref):
    x = x_ref[...]
    tw = tw_ref[...]
    tbr = tbr_ref[...]
    lab = lab_ref[...]
    sw = sw_ref[...]
    sbr = sbr_ref[...]
    slane = slane_ref[...]
    cadj = cadj_ref[...]
    sid = sid_ref[...]

    lane_l = lax.broadcasted_iota(jnp.int32, (_BB, DIM), 1)
    tb = jnp.sum(jnp.where(lane_l == jnp.bitwise_and(lab, 127), tbr, 0.0),
                 axis=1, keepdims=True)
    lane_s = lax.broadcasted_iota(jnp.int32, (S_PAD, DIM), 1)
    sb_col = jnp.sum(jnp.where(lane_s == slane, sbr, 0.0),
                     axis=1, keepdims=True)

    labf = lab.astype(jnp.float32)
    p_true = (jnp.log(labf + 2.0) - jnp.log(labf + 1.0)) / jnp.log(
        float(NUM_CLASSES + 1))
    exp_true = -_expm1_neg(NUM_SAMPLED * _log1m(p_true))
    tl = (jnp.sum(x * tw, axis=1, keepdims=True) + tb
          - jnp.log(exp_true))

    logits = lax.dot_general(
        x, sw, (((1,), (1,)), ((), ())),
        preferred_element_type=jnp.float32)
    ones_col = jnp.ones((_BB, 1), jnp.float32)
    logits = logits + lax.dot_general(
        ones_col, sb_col, (((1,), (1,)), ((), ())),
        preferred_element_type=jnp.float32)
    logits = logits + cadj
    logits = jnp.where(lab == sid, logits - 1e9, logits)

    m = jnp.maximum(jnp.max(logits, axis=1, keepdims=True), tl)
    ssum = (jnp.sum(jnp.exp(logits - m), axis=1, keepdims=True)
            + jnp.exp(tl - m))
    out_ref[...] = jnp.log(ssum) + m - tl


_tc_call = pl.pallas_call(
    _tc_body,
    grid=(BATCH // _BB,),
    in_specs=[
        pl.BlockSpec((_BB, DIM), lambda i: (i, 0)),
        pl.BlockSpec((_BB, DIM), lambda i: (i, 0)),
        pl.BlockSpec((_BB, DIM), lambda i: (i, 0)),
        pl.BlockSpec((_BB, 1), lambda i: (i, 0)),
        pl.BlockSpec((S_PAD, DIM), lambda i: (0, 0)),
        pl.BlockSpec((S_PAD, DIM), lambda i: (0, 0)),
        pl.BlockSpec((S_PAD, 1), lambda i: (0, 0)),
        pl.BlockSpec((1, S_PAD), lambda i: (0, 0)),
        pl.BlockSpec((1, S_PAD), lambda i: (0, 0)),
    ],
    out_specs=pl.BlockSpec((_BB, 1), lambda i: (i, 0)),
    out_shape=jax.ShapeDtypeStruct((BATCH, 1), jnp.float32),
)


def _sampled_constants():
    u = jax.random.uniform(jax.random.key(42), (NUM_SAMPLED,),
                           dtype=jnp.float32)
    ids = jnp.floor(jnp.exp(u * jnp.log(float(NUM_CLASSES + 1)))) - 1.0
    sampled = jnp.clip(ids, 0, NUM_CLASSES - 1).astype(jnp.int32)
    idf = sampled.astype(jnp.float32)
    p_samp = (jnp.log(idf + 2.0) - jnp.log(idf + 1.0)) / jnp.log(
        float(NUM_CLASSES + 1))
    exp_samp = -jnp.expm1(NUM_SAMPLED * jnp.log1p(-p_samp))
    cadj = -jnp.log(exp_samp)
    npad = S_PAD - NUM_SAMPLED
    sc_sid = jnp.concatenate([sampled, jnp.zeros((npad,), jnp.int32)])
    tc_sid = jnp.concatenate([sampled, jnp.full((npad,), -1, jnp.int32)])
    cadj_pad = jnp.concatenate([cadj, jnp.full((npad,), -1e30, jnp.float32)])
    return sc_sid, tc_sid, cadj_pad


def kernel(inputs, labels, W, b):
    sc_sid, tc_sid, cadj_pad = _sampled_constants()
    brs = jnp.pad(b, (0, _B_ROWS * DIM - NUM_CLASSES)).reshape(_B_ROWS, DIM)
    tw, sw, tbr, sbr = _sc_gather_kernel()(W, brs, labels, sc_sid)
    slane = jnp.bitwise_and(sc_sid, 127).reshape(S_PAD, 1)
    loss = _tc_call(inputs, tw, tbr, labels.reshape(BATCH, 1),
                    sw, sbr, slane, cadj_pad.reshape(1, S_PAD),
                    tc_sid.reshape(1, S_PAD))
    return loss.reshape(BATCH)
